# baseline (speedup 1.0000x reference)
import functools

import jax
import jax.numpy as jnp
import numpy as np
from jax import lax
from jax.experimental import pallas as pl
from jax.experimental.pallas import tpu as pltpu
from jax.experimental.pallas import tpu_sc as plsc

_MAX_MASK_SIZE = 27
_rng = np.random.RandomState(0)
_MASK = int(_rng.randint(0, _MAX_MASK_SIZE))
_START = int(_rng.randint(0, 128 - _MASK))
_END = _START + _MASK

_B, _F, _T = 128, 128, 2048

_NC, _NS = 2, 16
_NW = _NC * _NS
_BPW = _B // _NW

_CR = 16
_ROW_CHUNKS = [
    (0, 16), (16, 16), (32, 8),
    (64, 16), (80, 16), (96, 16), (112, 16),
]
_NCH = len(_ROW_CHUNKS)
_G5, _G6, _G7 = 40, 48, 56
_BCHUNKS = [(_G5, range(_START - _G5, 8)), (_G7, range(0, _END - _G7))]


def _sc_body(x_hbm, o_hbm, shared, bbuf0, bbuf1, zbuf,
             isem0, isem1, osem0, osem1, bisem0, bisem1, bosem0, bosem1,
             zsem):
    cid = lax.axis_index("c")
    sid = lax.axis_index("s")
    wid = sid * _NC + cid
    b0 = wid * _BPW

    chunks = []
    for b in range(_BPW):
        for r0, nr in _ROW_CHUNKS:
            chunks.append((b, r0, nr))
    n = len(chunks)
    isems = (isem0, isem1)
    osems = (osem0, osem1)

    def in_cp(i):
        b, r0, nr = chunks[i]
        return pltpu.make_async_copy(
            x_hbm.at[b0 + b, pl.ds(r0, nr), :],
            shared.at[sid, i % 2, pl.ds(0, nr), :],
            isems[i % 2])

    def out_cp(i):
        b, r0, nr = chunks[i]
        return pltpu.make_async_copy(
            shared.at[sid, i % 2, pl.ds(0, nr), :],
            o_hbm.at[b0 + b, pl.ds(r0, nr), :],
            osems[i % 2])

    bchunks = [(b, g0, zrows) for b in range(_BPW) for g0, zrows in _BCHUNKS]
    nb = len(bchunks)
    bbufs = (bbuf0, bbuf1)
    bisems = (bisem0, bisem1)
    bosems = (bosem0, bosem1)

    def bnd_in(j):
        b, g0, _ = bchunks[j]
        return pltpu.make_async_copy(
            x_hbm.at[b0 + b, pl.ds(g0, 8), :], bbufs[j % 2], bisems[j % 2])

    def bnd_out(j):
        b, g0, _ = bchunks[j]
        return pltpu.make_async_copy(
            bbufs[j % 2], o_hbm.at[b0 + b, pl.ds(g0, 8), :], bosems[j % 2])

    def g6_out(b):
        return pltpu.make_async_copy(
            zbuf, o_hbm.at[b0 + b, pl.ds(_G6, 8), :], zsem)

    def zero_rows(buf, rows):
        def zr(i, _):
            r = rows.start + i // (_T // 16)
            c = (i % (_T // 16)) * 16
            buf[r, pl.ds(c, 16)] = jnp.zeros((16,), jnp.float32)
            return 0
        lax.fori_loop(0, len(rows) * (_T // 16), zr, 0)

    def band_step(j):
        if j >= 1:
            bnd_out(j - 1).wait()
            if j + 1 < nb:
                bnd_in(j + 1).start()
        b, _, zrows = bchunks[j]
        bnd_in(j).wait()
        zero_rows(bbufs[j % 2], zrows)
        bnd_out(j).start()
        if j % 2 == 0:
            g6_out(b).start()

    bnd_in(0).start()
    bnd_in(1).start()
    in_cp(0).start()
    in_cp(1).start()
    zero_rows(zbuf, range(0, 8))

    bj = 0
    for i in range(n):
        in_cp(i).wait()
        out_cp(i).start()
        if i >= 1 and i + 1 < n:
            out_cp(i - 1).wait()
            in_cp(i + 1).start()
        if i % 2 == 1 and bj < nb:
            band_step(bj)
            bj += 1
    while bj < nb:
        band_step(bj)
        bj += 1
    out_cp(n - 2).wait()
    out_cp(n - 1).wait()
    bnd_out(nb - 1).wait()
    for b in range(_BPW):
        g6_out(b).wait()


def _sc_mask_copy(x):
    k = functools.partial(
        pl.kernel,
        mesh=plsc.VectorSubcoreMesh(core_axis_name="c", subcore_axis_name="s"),
        out_type=jax.ShapeDtypeStruct((_B, _F, _T), jnp.float32),
        scratch_types=[
            pltpu.VMEM_SHARED((_NS, 2, _CR, _T), jnp.float32),
            pltpu.VMEM((8, _T), jnp.float32),
            pltpu.VMEM((8, _T), jnp.float32),
            pltpu.VMEM((8, _T), jnp.float32),
        ] + [pltpu.SemaphoreType.DMA] * 9,
    )(_sc_body)
    return k(x)


def kernel(x):
    return _sc_mask_copy(x)


# speedup vs baseline: 1.0065x; 1.0065x over previous
import functools

import jax
import jax.numpy as jnp
import numpy as np
from jax import lax
from jax.experimental import pallas as pl
from jax.experimental.pallas import tpu as pltpu
from jax.experimental.pallas import tpu_sc as plsc

_MAX_MASK_SIZE = 27
_rng = np.random.RandomState(0)
_MASK = int(_rng.randint(0, _MAX_MASK_SIZE))
_START = int(_rng.randint(0, 128 - _MASK))
_END = _START + _MASK

_B, _F, _T = 128, 128, 2048

_NC, _NS = 2, 16
_NW = _NC * _NS
_BPW = _B // _NW

_CR = 16
_ROW_CHUNKS = [
    (0, 16), (16, 16), (32, 8),
    (64, 16), (80, 16), (96, 16), (112, 16),
]
_NCH = len(_ROW_CHUNKS)
_G5, _G6, _G7 = 40, 48, 56
_BCHUNKS = [(_G5, range(_START - _G5, 8)), (_G7, range(0, _END - _G7))]


def _sc_body(x_hbm, o_hbm, shared, bbuf0, bbuf1, zbuf,
             isem0, isem1, osem0, osem1, bisem0, bisem1, bosem0, bosem1,
             zsem):
    cid = lax.axis_index("c")
    sid = lax.axis_index("s")
    wid = sid * _NC + cid
    b0 = wid * _BPW

    chunks = []
    for b in range(_BPW):
        for r0, nr in _ROW_CHUNKS:
            chunks.append((b, r0, nr))
    n = len(chunks)
    isems = (isem0, isem1)
    osems = (osem0, osem1)

    def in_cp(i):
        b, r0, nr = chunks[i]
        return pltpu.make_async_copy(
            x_hbm.at[b0 + b, pl.ds(r0, nr), :],
            shared.at[sid, i % 2, pl.ds(0, nr), :],
            isems[i % 2])

    def out_cp(i):
        b, r0, nr = chunks[i]
        return pltpu.make_async_copy(
            shared.at[sid, i % 2, pl.ds(0, nr), :],
            o_hbm.at[b0 + b, pl.ds(r0, nr), :],
            osems[i % 2])

    bchunks = [(b, g0, zrows) for b in range(_BPW) for g0, zrows in _BCHUNKS]
    nb = len(bchunks)
    bbufs = (bbuf0, bbuf1)
    bisems = (bisem0, bisem1)
    bosems = (bosem0, bosem1)

    def bnd_in(j):
        b, g0, _ = bchunks[j]
        return pltpu.make_async_copy(
            x_hbm.at[b0 + b, pl.ds(g0, 8), :], bbufs[j % 2], bisems[j % 2])

    def bnd_out(j):
        b, g0, _ = bchunks[j]
        return pltpu.make_async_copy(
            bbufs[j % 2], o_hbm.at[b0 + b, pl.ds(g0, 8), :], bosems[j % 2])

    def g6_out(b):
        return pltpu.make_async_copy(
            zbuf, o_hbm.at[b0 + b, pl.ds(_G6, 8), :], zsem)

    def zero_rows(buf, rows):
        def zr(i, _):
            r = rows.start + i // (_T // 16)
            c = (i % (_T // 16)) * 16
            buf[r, pl.ds(c, 16)] = jnp.zeros((16,), jnp.float32)
            return 0
        lax.fori_loop(0, len(rows) * (_T // 16), zr, 0)

    def band_step(j):
        if j >= 1:
            bnd_out(j - 1).wait()
            if j + 1 < nb:
                bnd_in(j + 1).start()
        b, _, zrows = bchunks[j]
        bnd_in(j).wait()
        zero_rows(bbufs[j % 2], zrows)
        bnd_out(j).start()
        if j % 2 == 0:
            g6_out(b).start()

    bnd_in(0).start()
    bnd_in(1).start()
    in_cp(0).start()
    in_cp(1).start()
    zero_rows(zbuf, range(0, 8))

    bj = 0
    for i in range(n):
        in_cp(i).wait()
        out_cp(i).start()
        if i >= 1 and i + 1 < n:
            out_cp(i - 1).wait()
            in_cp(i + 1).start()
        if i % 3 == 2 and bj < nb:
            band_step(bj)
            bj += 1
    while bj < nb:
        band_step(bj)
        bj += 1
    out_cp(n - 2).wait()
    out_cp(n - 1).wait()
    bnd_out(nb - 1).wait()
    for b in range(_BPW):
        g6_out(b).wait()


def _sc_mask_copy(x):
    k = functools.partial(
        pl.kernel,
        mesh=plsc.VectorSubcoreMesh(core_axis_name="c", subcore_axis_name="s"),
        out_type=jax.ShapeDtypeStruct((_B, _F, _T), jnp.float32),
        scratch_types=[
            pltpu.VMEM_SHARED((_NS, 2, _CR, _T), jnp.float32),
            pltpu.VMEM((8, _T), jnp.float32),
            pltpu.VMEM((8, _T), jnp.float32),
            pltpu.VMEM((8, _T), jnp.float32),
        ] + [pltpu.SemaphoreType.DMA] * 9,
    )(_sc_body)
    return k(x)


def kernel(x):
    return _sc_mask_copy(x)


# speedup vs baseline: 1.0145x; 1.0079x over previous
import functools

import jax
import jax.numpy as jnp
import numpy as np
from jax import lax
from jax.experimental import pallas as pl
from jax.experimental.pallas import tpu as pltpu
from jax.experimental.pallas import tpu_sc as plsc

_MAX_MASK_SIZE = 27
_rng = np.random.RandomState(0)
_MASK = int(_rng.randint(0, _MAX_MASK_SIZE))
_START = int(_rng.randint(0, 128 - _MASK))
_END = _START + _MASK

_B, _F, _T = 128, 128, 2048

_NC, _NS = 2, 16
_NW = _NC * _NS
_BPW = _B // _NW

_CR = 16
_ROW_CHUNKS = [
    (0, 16), (16, 16), (32, 8),
    (64, 16), (80, 16), (96, 16), (112, 16),
]
_NCH = len(_ROW_CHUNKS)
_G5, _G6, _G7 = 40, 48, 56
_BCHUNKS = [(_G5, range(_START - _G5, 8)), (_G7, range(0, _END - _G7))]


def _sc_body(x_hbm, o_hbm, shared, bbuf0, bbuf1, bbuf2, zbuf,
             isem0, isem1, osem0, osem1, bisem0, bisem1, bisem2,
             bosem0, bosem1, bosem2, zsem):
    cid = lax.axis_index("c")
    sid = lax.axis_index("s")
    wid = sid * _NC + cid
    b0 = wid * _BPW

    chunks = []
    for b in range(_BPW):
        for r0, nr in _ROW_CHUNKS:
            chunks.append((b, r0, nr))
    n = len(chunks)
    isems = (isem0, isem1)
    osems = (osem0, osem1)

    def in_cp(i):
        b, r0, nr = chunks[i]
        return pltpu.make_async_copy(
            x_hbm.at[b0 + b, pl.ds(r0, nr), :],
            shared.at[sid, i % 2, pl.ds(0, nr), :],
            isems[i % 2])

    def out_cp(i):
        b, r0, nr = chunks[i]
        return pltpu.make_async_copy(
            shared.at[sid, i % 2, pl.ds(0, nr), :],
            o_hbm.at[b0 + b, pl.ds(r0, nr), :],
            osems[i % 2])

    bchunks = [(b, g0, zrows) for b in range(_BPW) for g0, zrows in _BCHUNKS]
    nb = len(bchunks)
    bbufs = (bbuf0, bbuf1, bbuf2)
    bisems = (bisem0, bisem1, bisem2)
    bosems = (bosem0, bosem1, bosem2)

    def bnd_in(j):
        b, g0, _ = bchunks[j]
        return pltpu.make_async_copy(
            x_hbm.at[b0 + b, pl.ds(g0, 8), :], bbufs[j % 3], bisems[j % 3])

    def bnd_out(j):
        b, g0, _ = bchunks[j]
        return pltpu.make_async_copy(
            bbufs[j % 3], o_hbm.at[b0 + b, pl.ds(g0, 8), :], bosems[j % 3])

    def g6_outs(b):
        return [pltpu.make_async_copy(
            zbuf, o_hbm.at[b0 + b, pl.ds(_G6, 8), pl.ds(h * 1024, 1024)],
            zsem) for h in range(2)]

    def zero_rows(buf, rows, cols):
        def zr(i, _):
            r = rows.start + i // (cols // 16)
            c = (i % (cols // 16)) * 16
            buf[r, pl.ds(c, 16)] = jnp.zeros((16,), jnp.float32)
            return 0
        lax.fori_loop(0, len(rows) * (cols // 16), zr, 0)

    def band_step(j):
        if j >= 2:
            bnd_out(j - 2).wait()
            if j + 1 < nb:
                bnd_in(j + 1).start()
        b, _, zrows = bchunks[j]
        bnd_in(j).wait()
        zero_rows(bbufs[j % 3], zrows, _T)
        bnd_out(j).start()
        if j % 2 == 0:
            for c in g6_outs(b):
                c.start()

    bnd_in(0).start()
    bnd_in(1).start()
    bnd_in(2).start()
    in_cp(0).start()
    in_cp(1).start()
    zero_rows(zbuf, range(0, 8), 1024)

    bj = 0
    for i in range(n):
        in_cp(i).wait()
        out_cp(i).start()
        if i >= 1 and i + 1 < n:
            out_cp(i - 1).wait()
            in_cp(i + 1).start()
        if i % 3 == 2 and bj < nb:
            band_step(bj)
            bj += 1
    while bj < nb:
        band_step(bj)
        bj += 1
    out_cp(n - 2).wait()
    out_cp(n - 1).wait()
    bnd_out(nb - 2).wait()
    bnd_out(nb - 1).wait()
    for b in range(_BPW):
        for c in g6_outs(b):
            c.wait()


def _sc_mask_copy(x):
    k = functools.partial(
        pl.kernel,
        mesh=plsc.VectorSubcoreMesh(core_axis_name="c", subcore_axis_name="s"),
        out_type=jax.ShapeDtypeStruct((_B, _F, _T), jnp.float32),
        scratch_types=[
            pltpu.VMEM_SHARED((_NS, 2, _CR, _T), jnp.float32),
            pltpu.VMEM((8, _T), jnp.float32),
            pltpu.VMEM((8, _T), jnp.float32),
            pltpu.VMEM((8, _T), jnp.float32),
            pltpu.VMEM((8, 1024), jnp.float32),
        ] + [pltpu.SemaphoreType.DMA] * 11,
    )(_sc_body)
    return k(x)


def kernel(x):
    return _sc_mask_copy(x)
